# Initial kernel scaffold; baseline (speedup 1.0000x reference)
import jax
import jax.numpy as jnp
from jax.experimental import pallas as pl


def kernel(pred_logits, pred_boxes, target_sizes):
    raise NotImplementedError("write your pallas kernel here")


# speedup vs baseline: 2.7720x; 2.7720x over previous
import functools

import jax
import jax.numpy as jnp
from jax import lax
from jax.experimental import pallas as pl
from jax.experimental.pallas import tpu as pltpu
from jax.experimental.pallas import tpu_sc as plsc

K_SEL = 120
N_REAL = 81900
N_PAD = 81920
NB = 128
QN = 900
CN = 91
NW = 32
BPW = NB // NW
CAP_A = 8208
CAP_B = 2064
INT_MIN = -(2 ** 31)


def _digit(kv, shift):
    if shift == 24:
        return (kv >> 24) + 128
    return (kv >> shift) & 0xFF


def _sc_body(keys_hbm, boxes_hbm, scale_hbm, sc_out, lb_out, bx_out,
             data, ca_k, ca_i, cb_k, cb_i, histL, hist1,
             win_k, win_i, srt_k, srt_i, boxbuf, scalebuf,
             sc_buf, lb_buf, bx_buf):
    wid = lax.axis_index("s") * 2 + lax.axis_index("c")
    iota = lax.broadcasted_iota(jnp.int32, (16,), 0)
    ioff = iota * 256
    zeros16 = jnp.zeros((16,), jnp.int32)
    ones16 = jnp.ones((16,), jnp.int32)

    def hist_round(load_fn, n_src, shift, k_need):
        def zh(m, c):
            histL[pl.ds(m * 16, 16)] = zeros16
            return c
        lax.fori_loop(0, 256, zh, 0)

        nv = (n_src + 15) // 16

        def hb(i, c):
            kv, _, valid = load_fn(i)
            d = _digit(kv, shift)
            plsc.addupdate_scatter(histL, [ioff + d], ones16, mask=valid)
            return c
        lax.fori_loop(0, nv, hb, 0)

        def rb(g, c):
            acc = zeros16
            for l in range(16):
                acc = acc + histL[pl.ds(l * 256 + g * 16, 16)]
            hist1[pl.ds(g * 16, 16)] = acc
            return c
        lax.fori_loop(0, 16, rb, 0)

        def sb(t, carry):
            bsel, acc = carry
            g = 15 - t
            v = hist1[pl.ds(g * 16, 16)]
            suff = lax.rev(plsc.cumsum(lax.rev(v, (0,))), (0,))
            cnt_ge = acc + suff
            dv = jnp.where(cnt_ge >= k_need, g * 16 + iota, -1)
            bsel = jnp.maximum(bsel, jnp.max(dv))
            return bsel, acc + jnp.sum(v)
        bsel, _ = lax.fori_loop(0, 16, sb, (jnp.int32(-1), jnp.int32(0)))

        def cgb(g, acc):
            v = hist1[pl.ds(g * 16, 16)]
            return acc + jnp.sum(jnp.where((g * 16 + iota) > bsel, v, 0))
        cntgt = lax.fori_loop(0, 16, cgb, jnp.int32(0))
        return bsel, cntgt

    def collect_round(load_fn, n_src, shift, bsel, dst_k, dst_i, cap,
                      woff, last, k_take):
        nv = (n_src + 15) // 16

        def cb(i, carry):
            woff, coff = carry
            kv, idxv, valid = load_fn(i)
            d = _digit(kv, shift)
            wm = (d > bsel) & valid
            em = (d == bsel) & valid
            plsc.store_compressed(win_k.at[pl.ds(woff, 16)], kv, mask=wm)
            plsc.store_compressed(win_i.at[pl.ds(woff, 16)], idxv, mask=wm)
            woff = woff + jnp.sum(wm)
            if last:
                r = coff + plsc.cumsum(em.astype(jnp.int32)) - 1
                tm = em & (r < k_take)
                plsc.store_compressed(win_k.at[pl.ds(woff, 16)], kv, mask=tm)
                plsc.store_compressed(win_i.at[pl.ds(woff, 16)], idxv, mask=tm)
                woff = woff + jnp.sum(tm)
                coff = coff + jnp.sum(em)
            else:
                em = em & (coff < cap - 16)
                plsc.store_compressed(dst_k.at[pl.ds(coff, 16)], kv, mask=em)
                plsc.store_compressed(dst_i.at[pl.ds(coff, 16)], idxv, mask=em)
                coff = coff + jnp.sum(em)
            return woff, coff
        return lax.fori_loop(0, nv, cb, (woff, jnp.int32(0)))

    def per_batch(t, carry):
        b = wid * BPW + t
        pltpu.sync_copy(keys_hbm.at[pl.ds(b * N_PAD, N_PAD)], data)
        pltpu.sync_copy(boxes_hbm.at[pl.ds(b * (QN * 4), QN * 4)],
                        boxbuf.at[pl.ds(0, QN * 4)])
        pltpu.sync_copy(scale_hbm.at[pl.ds(b * 32, 32)], scalebuf)

        win_k[pl.ds(112, 16)] = jnp.full((16,), INT_MIN, jnp.int32)
        win_i[pl.ds(112, 16)] = N_PAD + iota

        def load_data(i):
            kv = data[pl.ds(i * 16, 16)]
            return kv, i * 16 + iota, (iota < 16)

        def mk_load(src_k, src_i, n):
            def f(i):
                kv = src_k[pl.ds(i * 16, 16)]
                iv = src_i[pl.ds(i * 16, 16)]
                return kv, iv, (i * 16 + iota) < n
            return f

        k1 = jnp.int32(K_SEL)
        b1, cg1 = hist_round(load_data, N_PAD, 24, k1)
        woff, nA = collect_round(load_data, N_PAD, 24, b1, ca_k, ca_i, CAP_A,
                                 jnp.int32(0), False, 0)

        k2 = k1 - cg1
        loadA = mk_load(ca_k, ca_i, nA)
        b2, cg2 = hist_round(loadA, nA, 16, k2)
        woff, nB = collect_round(loadA, nA, 16, b2, cb_k, cb_i, CAP_B,
                                 woff, False, 0)

        k3 = k2 - cg2
        loadB = mk_load(cb_k, cb_i, nB)
        b3, cg3 = hist_round(loadB, nB, 8, k3)
        woff, nA2 = collect_round(loadB, nB, 8, b3, ca_k, ca_i, CAP_A,
                                  woff, False, 0)

        k4 = k3 - cg3
        loadA2 = mk_load(ca_k, ca_i, nA2)
        b4, cg4 = hist_round(loadA2, nA2, 0, k4)
        woff, _ = collect_round(loadA2, nA2, 0, b4, None, None, 0,
                                woff, True, k4 - cg4)

        def rank_i(i, c):
            ki = win_k[pl.ds(i * 16, 16)]
            ii = win_i[pl.ds(i * 16, 16)]

            def rank_m(m, acc):
                idxv = (m & ~15) + ((iota + m) & 15)
                kr = plsc.load_gather(win_k, [idxv])
                ir = plsc.load_gather(win_i, [idxv])
                beat = (kr > ki) | ((kr == ki) & (ir < ii))
                return acc + beat.astype(jnp.int32)
            pos = lax.fori_loop(0, 128, rank_m, zeros16)
            plsc.store_scatter(srt_k, [pos], ki)
            plsc.store_scatter(srt_i, [pos], ii)
            return c
        lax.fori_loop(0, 8, rank_i, 0)

        W = scalebuf[pl.ds(0, 16)]
        H = scalebuf[pl.ds(16, 16)]

        def out_v(v, c):
            kk = srt_k[pl.ds(v * 16, 16)]
            id2 = srt_i[pl.ds(v * 16, 16)]
            bits = jnp.where(kk >= 0, kk, kk ^ 0x7FFFFFFF)
            val = plsc.bitcast(bits, jnp.float32)
            sc_buf[pl.ds(v * 16, 16)] = 1.0 / (1.0 + jnp.exp(-val))
            lb_buf[pl.ds(v * 16, 16)] = id2 % CN
            q4 = jnp.minimum(id2 // CN, QN - 1) * 4
            cx = plsc.load_gather(boxbuf, [q4])
            cy = plsc.load_gather(boxbuf, [q4 + 1])
            w = plsc.load_gather(boxbuf, [q4 + 2])
            h = plsc.load_gather(boxbuf, [q4 + 3])
            bx_buf[pl.ds(v * 16, 16)] = (cx - 0.5 * w) * W
            bx_buf[pl.ds(128 + v * 16, 16)] = (cy - 0.5 * h) * H
            bx_buf[pl.ds(256 + v * 16, 16)] = (cx + 0.5 * w) * W
            bx_buf[pl.ds(384 + v * 16, 16)] = (cy + 0.5 * h) * H
            return c
        lax.fori_loop(0, 8, out_v, 0)

        pltpu.sync_copy(sc_buf, sc_out.at[pl.ds(b * 128, 128)])
        pltpu.sync_copy(lb_buf, lb_out.at[pl.ds(b * 128, 128)])
        pltpu.sync_copy(bx_buf, bx_out.at[pl.ds(b * 512, 512)])
        return carry

    lax.fori_loop(0, BPW, per_batch, 0)


@jax.jit
def _post_process_sc(keys_flat, boxes_flat, scale_flat):
    mesh = plsc.VectorSubcoreMesh(core_axis_name="c", subcore_axis_name="s")
    f = pl.kernel(
        _sc_body,
        out_type=(
            jax.ShapeDtypeStruct((NB * 128,), jnp.float32),
            jax.ShapeDtypeStruct((NB * 128,), jnp.int32),
            jax.ShapeDtypeStruct((NB * 512,), jnp.float32),
        ),
        mesh=mesh,
        compiler_params=pltpu.CompilerParams(needs_layout_passes=False),
        scratch_types=[
            pltpu.VMEM((N_PAD,), jnp.int32),
            pltpu.VMEM((CAP_A,), jnp.int32),
            pltpu.VMEM((CAP_A,), jnp.int32),
            pltpu.VMEM((CAP_B,), jnp.int32),
            pltpu.VMEM((CAP_B,), jnp.int32),
            pltpu.VMEM((4096,), jnp.int32),
            pltpu.VMEM((256,), jnp.int32),
            pltpu.VMEM((160,), jnp.int32),
            pltpu.VMEM((160,), jnp.int32),
            pltpu.VMEM((128,), jnp.int32),
            pltpu.VMEM((128,), jnp.int32),
            pltpu.VMEM((3616,), jnp.float32),
            pltpu.VMEM((32,), jnp.float32),
            pltpu.VMEM((128,), jnp.float32),
            pltpu.VMEM((128,), jnp.int32),
            pltpu.VMEM((512,), jnp.float32),
        ],
    )
    return f(keys_flat, boxes_flat, scale_flat)


def kernel(pred_logits, pred_boxes, target_sizes):
    B, Q, C = pred_logits.shape
    flat = pred_logits.reshape(B, Q * C)
    bits = lax.bitcast_convert_type(flat, jnp.int32)
    keys = jnp.where(bits >= 0, bits, bits ^ 0x7FFFFFFF)
    keys = jnp.pad(keys, ((0, 0), (0, N_PAD - N_REAL)),
                   constant_values=INT_MIN)
    ts = target_sizes.astype(jnp.float32)
    scale = jnp.concatenate(
        [jnp.tile(ts[:, 1:2], (1, 16)), jnp.tile(ts[:, 0:1], (1, 16))], axis=1)
    sc_f, lb_f, bx_f = _post_process_sc(
        keys.reshape(-1), pred_boxes.reshape(-1), scale.reshape(-1))
    scores = sc_f.reshape(B, 128)[:, :K_SEL]
    labels = lb_f.reshape(B, 128)[:, :K_SEL]
    boxes = bx_f.reshape(B, 4, 128)[:, :, :K_SEL].transpose(0, 2, 1)
    return scores, labels, boxes


# speedup vs baseline: 3.0629x; 1.1049x over previous
import functools

import jax
import jax.numpy as jnp
from jax import lax
from jax.experimental import pallas as pl
from jax.experimental.pallas import tpu as pltpu
from jax.experimental.pallas import tpu_sc as plsc

K_SEL = 120
N_REAL = 81900
N_PAD = 81920
NB = 128
QN = 900
CN = 91
NW = 32
BPW = NB // NW
CAP_A = 8208
CAP_B = 4112
INT_MIN = -(2 ** 31)
UH = 8
NV = N_PAD // 16


def _sc_body(keys_hbm, boxes_hbm, scale_hbm, sc_out, lb_out, bx_out,
             data, ca_k, ca_i, cb_k, cb_i, histL, hist1,
             win_k, win_i, srt_k, srt_i, boxbuf, scalebuf,
             sc_buf, lb_buf, bx_buf):
    wid = lax.axis_index("s") * 2 + lax.axis_index("c")
    iota = lax.broadcasted_iota(jnp.int32, (16,), 0)
    ioff = iota * 256
    zeros16 = jnp.zeros((16,), jnp.int32)
    ones16 = jnp.ones((16,), jnp.int32)

    def zero_hist():
        def zh(m, c):
            for u in range(UH):
                histL[pl.ds((m * UH + u) * 16, 16)] = zeros16
            return c
        lax.fori_loop(0, 256 // UH, zh, 0)

    def select_digit(k_need):
        def rb(g, c):
            acc = zeros16
            for l in range(16):
                acc = acc + histL[pl.ds(l * 256 + g * 16, 16)]
            hist1[pl.ds(g * 16, 16)] = acc
            return c
        lax.fori_loop(0, 16, rb, 0)

        def sb(t, carry):
            bsel, acc = carry
            g = 15 - t
            v = hist1[pl.ds(g * 16, 16)]
            suff = lax.rev(plsc.cumsum(lax.rev(v, (0,))), (0,))
            cnt_ge = acc + suff
            dv = jnp.where(cnt_ge >= k_need, g * 16 + iota, -1)
            bsel = jnp.maximum(bsel, jnp.max(dv))
            return bsel, acc + jnp.sum(v)
        bsel, _ = lax.fori_loop(0, 16, sb, (jnp.int32(-1), jnp.int32(0)))

        def cgb(g, acc):
            v = hist1[pl.ds(g * 16, 16)]
            return acc + jnp.sum(jnp.where((g * 16 + iota) > bsel, v, 0))
        cntgt = lax.fori_loop(0, 16, cgb, jnp.int32(0))
        return bsel, cntgt

    def mid_round(src_k, n_src, shift, base, k_r):
        zero_hist()
        himax = base + ((1 << (shift + 8)) - 1)

        def hb(i, c):
            kv = src_k[pl.ds(i * 16, 16)]
            m = (kv <= himax) & ((i * 16 + iota) < n_src)
            d = (kv >> shift) & 0xFF
            plsc.addupdate_scatter(histL, [ioff + d], ones16, mask=m)
            return c
        lax.fori_loop(0, (n_src + 15) // 16, hb, 0)
        bsel, cg = select_digit(k_r)
        return base + (bsel << shift), cg

    def mid_collect(src_k, src_i, dst_k, dst_i, n_src, base, cap):
        def cb(i, coff):
            kv = src_k[pl.ds(i * 16, 16)]
            iv = src_i[pl.ds(i * 16, 16)]
            em = ((kv >= base) & ((i * 16 + iota) < n_src)
                  & (coff < cap - 16))
            plsc.store_compressed(dst_k.at[pl.ds(coff, 16)], kv, mask=em)
            plsc.store_compressed(dst_i.at[pl.ds(coff, 16)], iv, mask=em)
            return coff + jnp.sum(em)
        return lax.fori_loop(0, (n_src + 15) // 16, cb, jnp.int32(0))

    def per_batch(t, carry):
        b = wid * BPW + t
        pltpu.sync_copy(keys_hbm.at[pl.ds(b * N_PAD, N_PAD)], data)
        pltpu.sync_copy(boxes_hbm.at[pl.ds(b * (QN * 4), QN * 4)],
                        boxbuf.at[pl.ds(0, QN * 4)])
        pltpu.sync_copy(scale_hbm.at[pl.ds(b * 32, 32)], scalebuf)

        win_k[pl.ds(112, 16)] = jnp.full((16,), INT_MIN, jnp.int32)
        win_i[pl.ds(112, 16)] = N_PAD + iota

        zero_hist()

        def h1(i, c):
            for u in range(UH):
                kv = data[pl.ds((i * UH + u) * 16, 16)]
                d = (kv >> 24) + 128
                plsc.addupdate_scatter(histL, [ioff + d], ones16)
            return c
        lax.fori_loop(0, NV // UH, h1, 0)
        b1, cg1 = select_digit(jnp.int32(K_SEL))
        base = (b1 - 128) << 24

        def c1(i, coff):
            for u in range(UH):
                i16 = (i * UH + u) * 16
                kv = data[pl.ds(i16, 16)]
                em = (kv >= base) & (coff < CAP_A - 16)
                plsc.store_compressed(ca_k.at[pl.ds(coff, 16)], kv, mask=em)
                plsc.store_compressed(ca_i.at[pl.ds(coff, 16)], i16 + iota,
                                      mask=em)
                coff = coff + jnp.sum(em)
            return coff
        c1n = lax.fori_loop(0, NV // UH, c1, jnp.int32(0))

        k2 = K_SEL - cg1
        base, cg2 = mid_round(ca_k, c1n, 16, base, k2)
        c2n = mid_collect(ca_k, ca_i, cb_k, cb_i, c1n, base, CAP_B)
        k3 = k2 - cg2
        base, cg3 = mid_round(cb_k, c2n, 8, base, k3)
        c3n = mid_collect(cb_k, cb_i, ca_k, ca_i, c2n, base, CAP_A)
        k4 = k3 - cg3
        tkey, cg4 = mid_round(ca_k, c3n, 0, base, k4)
        k5 = k4 - cg4

        def fb(i, carry):
            woff, cseen = carry
            kv = ca_k[pl.ds(i * 16, 16)]
            iv = ca_i[pl.ds(i * 16, 16)]
            valid = (i * 16 + iota) < c3n
            wm = (kv > tkey) & valid
            plsc.store_compressed(win_k.at[pl.ds(woff, 16)], kv, mask=wm)
            plsc.store_compressed(win_i.at[pl.ds(woff, 16)], iv, mask=wm)
            woff = woff + jnp.sum(wm)
            em = (kv == tkey) & valid
            r = cseen + plsc.cumsum(em.astype(jnp.int32)) - 1
            tm = em & (r < k5)
            plsc.store_compressed(win_k.at[pl.ds(woff, 16)], kv, mask=tm)
            plsc.store_compressed(win_i.at[pl.ds(woff, 16)], iv, mask=tm)
            return woff + jnp.sum(tm), cseen + jnp.sum(em)
        lax.fori_loop(0, (c3n + 15) // 16, fb,
                      (jnp.int32(0), jnp.int32(0)))

        def rank_i(i, c):
            ki = win_k[pl.ds(i * 16, 16)]
            ii = win_i[pl.ds(i * 16, 16)]

            def rank_m(m, acc):
                for u in range(4):
                    mm = m * 4 + u
                    idxv = ((mm >> 4) << 4) + ((iota + mm) & 15)
                    kr = plsc.load_gather(win_k, [idxv])
                    ir = plsc.load_gather(win_i, [idxv])
                    beat = (kr > ki) | ((kr == ki) & (ir < ii))
                    acc = acc + beat.astype(jnp.int32)
                return acc
            pos = lax.fori_loop(0, 32, rank_m, zeros16)
            plsc.store_scatter(srt_k, [pos], ki)
            plsc.store_scatter(srt_i, [pos], ii)
            return c
        lax.fori_loop(0, 8, rank_i, 0)

        W = scalebuf[pl.ds(0, 16)]
        H = scalebuf[pl.ds(16, 16)]

        def out_v(v, c):
            kk = srt_k[pl.ds(v * 16, 16)]
            id2 = srt_i[pl.ds(v * 16, 16)]
            bits = jnp.where(kk >= 0, kk, kk ^ 0x7FFFFFFF)
            val = plsc.bitcast(bits, jnp.float32)
            sc_buf[pl.ds(v * 16, 16)] = 1.0 / (1.0 + jnp.exp(-val))
            lb_buf[pl.ds(v * 16, 16)] = id2 % CN
            q4 = jnp.minimum(id2 // CN, QN - 1) * 4
            cx = plsc.load_gather(boxbuf, [q4])
            cy = plsc.load_gather(boxbuf, [q4 + 1])
            w = plsc.load_gather(boxbuf, [q4 + 2])
            h = plsc.load_gather(boxbuf, [q4 + 3])
            bx_buf[pl.ds(v * 16, 16)] = (cx - 0.5 * w) * W
            bx_buf[pl.ds(128 + v * 16, 16)] = (cy - 0.5 * h) * H
            bx_buf[pl.ds(256 + v * 16, 16)] = (cx + 0.5 * w) * W
            bx_buf[pl.ds(384 + v * 16, 16)] = (cy + 0.5 * h) * H
            return c
        lax.fori_loop(0, 8, out_v, 0)

        pltpu.sync_copy(sc_buf, sc_out.at[pl.ds(b * 128, 128)])
        pltpu.sync_copy(lb_buf, lb_out.at[pl.ds(b * 128, 128)])
        pltpu.sync_copy(bx_buf, bx_out.at[pl.ds(b * 512, 512)])
        return carry

    lax.fori_loop(0, BPW, per_batch, 0)


@jax.jit
def _post_process_sc(keys_flat, boxes_flat, scale_flat):
    mesh = plsc.VectorSubcoreMesh(core_axis_name="c", subcore_axis_name="s")
    f = pl.kernel(
        _sc_body,
        out_type=(
            jax.ShapeDtypeStruct((NB * 128,), jnp.float32),
            jax.ShapeDtypeStruct((NB * 128,), jnp.int32),
            jax.ShapeDtypeStruct((NB * 512,), jnp.float32),
        ),
        mesh=mesh,
        compiler_params=pltpu.CompilerParams(needs_layout_passes=False),
        scratch_types=[
            pltpu.VMEM((N_PAD,), jnp.int32),
            pltpu.VMEM((CAP_A,), jnp.int32),
            pltpu.VMEM((CAP_A,), jnp.int32),
            pltpu.VMEM((CAP_B,), jnp.int32),
            pltpu.VMEM((CAP_B,), jnp.int32),
            pltpu.VMEM((4096,), jnp.int32),
            pltpu.VMEM((256,), jnp.int32),
            pltpu.VMEM((160,), jnp.int32),
            pltpu.VMEM((160,), jnp.int32),
            pltpu.VMEM((128,), jnp.int32),
            pltpu.VMEM((128,), jnp.int32),
            pltpu.VMEM((3616,), jnp.float32),
            pltpu.VMEM((32,), jnp.float32),
            pltpu.VMEM((128,), jnp.float32),
            pltpu.VMEM((128,), jnp.int32),
            pltpu.VMEM((512,), jnp.float32),
        ],
    )
    return f(keys_flat, boxes_flat, scale_flat)


def kernel(pred_logits, pred_boxes, target_sizes):
    B, Q, C = pred_logits.shape
    flat = pred_logits.reshape(B, Q * C)
    bits = lax.bitcast_convert_type(flat, jnp.int32)
    keys = jnp.where(bits >= 0, bits, bits ^ 0x7FFFFFFF)
    keys = jnp.pad(keys, ((0, 0), (0, N_PAD - N_REAL)),
                   constant_values=INT_MIN)
    ts = target_sizes.astype(jnp.float32)
    scale = jnp.concatenate(
        [jnp.tile(ts[:, 1:2], (1, 16)), jnp.tile(ts[:, 0:1], (1, 16))], axis=1)
    sc_f, lb_f, bx_f = _post_process_sc(
        keys.reshape(-1), pred_boxes.reshape(-1), scale.reshape(-1))
    scores = sc_f.reshape(B, 128)[:, :K_SEL]
    labels = lb_f.reshape(B, 128)[:, :K_SEL]
    boxes = bx_f.reshape(B, 4, 128)[:, :, :K_SEL].transpose(0, 2, 1)
    return scores, labels, boxes


# speedup vs baseline: 3.1092x; 1.0151x over previous
import functools

import jax
import jax.numpy as jnp
from jax import lax
from jax.experimental import pallas as pl
from jax.experimental.pallas import tpu as pltpu
from jax.experimental.pallas import tpu_sc as plsc

K_SEL = 120
N_REAL = 81900
N_PAD = 81920
NB = 128
QN = 900
CN = 91
NW = 32
BPW = NB // NW
CAP_A = 8208
CAP_B = 4112
INT_MIN = -(2 ** 31)
UH = 8
NV = N_PAD // 16


def _sc_body(keys_hbm, boxes_hbm, scale_hbm, sc_out, lb_out, bx_out,
             data, ca_k, ca_i, cb_k, cb_i, histL, hist1,
             win_k, win_i, srt_k, srt_i, boxbuf, scalebuf,
             sc_buf, lb_buf, bx_buf):
    wid = lax.axis_index("s") * 2 + lax.axis_index("c")
    iota = lax.broadcasted_iota(jnp.int32, (16,), 0)
    iota17 = iota * 17
    zeros16 = jnp.zeros((16,), jnp.int32)
    ones16 = jnp.ones((16,), jnp.int32)

    def zero_hist():
        def zh(m, c):
            for u in range(UH):
                histL[pl.ds((m * UH + u) * 16, 16)] = zeros16
            return c
        lax.fori_loop(0, 4352 // (16 * UH), zh, 0)

    def select_digit(k_need):
        def rb(g, c):
            acc = zeros16
            gbase = g * 272 + iota17
            for j in range(16):
                acc = acc + plsc.load_gather(histL, [gbase + j])
            hist1[pl.ds(g * 16, 16)] = acc
            return c
        lax.fori_loop(0, 16, rb, 0)

        def sb(t, carry):
            bsel, acc = carry
            g = 15 - t
            v = hist1[pl.ds(g * 16, 16)]
            suff = lax.rev(plsc.cumsum(lax.rev(v, (0,))), (0,))
            cnt_ge = acc + suff
            dv = jnp.where(cnt_ge >= k_need, g * 16 + iota, -1)
            bsel = jnp.maximum(bsel, jnp.max(dv))
            return bsel, acc + jnp.sum(v)
        bsel, _ = lax.fori_loop(0, 16, sb, (jnp.int32(-1), jnp.int32(0)))

        def cgb(g, acc):
            v = hist1[pl.ds(g * 16, 16)]
            return acc + jnp.sum(jnp.where((g * 16 + iota) > bsel, v, 0))
        cntgt = lax.fori_loop(0, 16, cgb, jnp.int32(0))
        return bsel, cntgt

    def mid_round(src_k, n_src, shift, base, k_r):
        zero_hist()
        himax = base + ((1 << (shift + 8)) - 1)

        def hb(i, c):
            kv = src_k[pl.ds(i * 16, 16)]
            m = (kv <= himax) & ((i * 16 + iota) < n_src)
            d = (kv >> shift) & 0xFF
            plsc.addupdate_scatter(histL, [d * 17 + iota], ones16, mask=m)
            return c
        lax.fori_loop(0, (n_src + 15) // 16, hb, 0)
        bsel, cg = select_digit(k_r)
        return base + (bsel << shift), cg

    def mid_collect(src_k, src_i, dst_k, dst_i, n_src, base, cap):
        def cb(i, coff):
            kv = src_k[pl.ds(i * 16, 16)]
            iv = src_i[pl.ds(i * 16, 16)]
            em = ((kv >= base) & ((i * 16 + iota) < n_src)
                  & (coff < cap - 16))
            plsc.store_compressed(dst_k.at[pl.ds(coff, 16)], kv, mask=em)
            plsc.store_compressed(dst_i.at[pl.ds(coff, 16)], iv, mask=em)
            return coff + jnp.sum(em)
        return lax.fori_loop(0, (n_src + 15) // 16, cb, jnp.int32(0))

    def per_batch(t, carry):
        b = wid * BPW + t
        pltpu.sync_copy(keys_hbm.at[pl.ds(b * N_PAD, N_PAD)], data)
        pltpu.sync_copy(boxes_hbm.at[pl.ds(b * (QN * 4), QN * 4)],
                        boxbuf.at[pl.ds(0, QN * 4)])
        pltpu.sync_copy(scale_hbm.at[pl.ds(b * 32, 32)], scalebuf)

        win_k[pl.ds(112, 16)] = jnp.full((16,), INT_MIN, jnp.int32)
        win_i[pl.ds(112, 16)] = N_PAD + iota

        zero_hist()

        def h1(i, c):
            for u in range(UH):
                kv = data[pl.ds((i * UH + u) * 16, 16)]
                d = (kv >> 24) + 128
                plsc.addupdate_scatter(histL, [d * 17 + iota], ones16)
            return c
        lax.fori_loop(0, NV // UH, h1, 0)
        b1, cg1 = select_digit(jnp.int32(K_SEL))
        base = (b1 - 128) << 24

        def c1(i, coff):
            for u in range(UH):
                i16 = (i * UH + u) * 16
                kv = data[pl.ds(i16, 16)]
                em = (kv >= base) & (coff < CAP_A - 16)
                plsc.store_compressed(ca_k.at[pl.ds(coff, 16)], kv, mask=em)
                plsc.store_compressed(ca_i.at[pl.ds(coff, 16)], i16 + iota,
                                      mask=em)
                coff = coff + jnp.sum(em)
            return coff
        c1n = lax.fori_loop(0, NV // UH, c1, jnp.int32(0))

        k2 = K_SEL - cg1
        base, cg2 = mid_round(ca_k, c1n, 16, base, k2)
        c2n = mid_collect(ca_k, ca_i, cb_k, cb_i, c1n, base, CAP_B)
        k3 = k2 - cg2
        base, cg3 = mid_round(cb_k, c2n, 8, base, k3)
        c3n = mid_collect(cb_k, cb_i, ca_k, ca_i, c2n, base, CAP_A)
        k4 = k3 - cg3
        tkey, cg4 = mid_round(ca_k, c3n, 0, base, k4)
        k5 = k4 - cg4

        def fb(i, carry):
            woff, cseen = carry
            kv = ca_k[pl.ds(i * 16, 16)]
            iv = ca_i[pl.ds(i * 16, 16)]
            valid = (i * 16 + iota) < c3n
            wm = (kv > tkey) & valid
            plsc.store_compressed(win_k.at[pl.ds(woff, 16)], kv, mask=wm)
            plsc.store_compressed(win_i.at[pl.ds(woff, 16)], iv, mask=wm)
            woff = woff + jnp.sum(wm)
            em = (kv == tkey) & valid
            r = cseen + plsc.cumsum(em.astype(jnp.int32)) - 1
            tm = em & (r < k5)
            plsc.store_compressed(win_k.at[pl.ds(woff, 16)], kv, mask=tm)
            plsc.store_compressed(win_i.at[pl.ds(woff, 16)], iv, mask=tm)
            return woff + jnp.sum(tm), cseen + jnp.sum(em)
        lax.fori_loop(0, (c3n + 15) // 16, fb,
                      (jnp.int32(0), jnp.int32(0)))

        def rank_i(i, c):
            ki = win_k[pl.ds(i * 16, 16)]
            ii = win_i[pl.ds(i * 16, 16)]

            def rank_m(m, acc):
                for u in range(4):
                    mm = m * 4 + u
                    idxv = ((mm >> 4) << 4) + ((iota + mm) & 15)
                    kr = plsc.load_gather(win_k, [idxv])
                    ir = plsc.load_gather(win_i, [idxv])
                    beat = (kr > ki) | ((kr == ki) & (ir < ii))
                    acc = acc + beat.astype(jnp.int32)
                return acc
            pos = lax.fori_loop(0, 32, rank_m, zeros16)
            plsc.store_scatter(srt_k, [pos], ki)
            plsc.store_scatter(srt_i, [pos], ii)
            return c
        lax.fori_loop(0, 8, rank_i, 0)

        W = scalebuf[pl.ds(0, 16)]
        H = scalebuf[pl.ds(16, 16)]

        def out_v(v, c):
            kk = srt_k[pl.ds(v * 16, 16)]
            id2 = srt_i[pl.ds(v * 16, 16)]
            bits = jnp.where(kk >= 0, kk, kk ^ 0x7FFFFFFF)
            val = plsc.bitcast(bits, jnp.float32)
            sc_buf[pl.ds(v * 16, 16)] = 1.0 / (1.0 + jnp.exp(-val))
            lb_buf[pl.ds(v * 16, 16)] = id2 % CN
            q4 = jnp.minimum(id2 // CN, QN - 1) * 4
            cx = plsc.load_gather(boxbuf, [q4])
            cy = plsc.load_gather(boxbuf, [q4 + 1])
            w = plsc.load_gather(boxbuf, [q4 + 2])
            h = plsc.load_gather(boxbuf, [q4 + 3])
            bx_buf[pl.ds(v * 16, 16)] = (cx - 0.5 * w) * W
            bx_buf[pl.ds(128 + v * 16, 16)] = (cy - 0.5 * h) * H
            bx_buf[pl.ds(256 + v * 16, 16)] = (cx + 0.5 * w) * W
            bx_buf[pl.ds(384 + v * 16, 16)] = (cy + 0.5 * h) * H
            return c
        lax.fori_loop(0, 8, out_v, 0)

        pltpu.sync_copy(sc_buf, sc_out.at[pl.ds(b * 128, 128)])
        pltpu.sync_copy(lb_buf, lb_out.at[pl.ds(b * 128, 128)])
        pltpu.sync_copy(bx_buf, bx_out.at[pl.ds(b * 512, 512)])
        return carry

    lax.fori_loop(0, BPW, per_batch, 0)


@jax.jit
def _post_process_sc(keys_flat, boxes_flat, scale_flat):
    mesh = plsc.VectorSubcoreMesh(core_axis_name="c", subcore_axis_name="s")
    f = pl.kernel(
        _sc_body,
        out_type=(
            jax.ShapeDtypeStruct((NB * 128,), jnp.float32),
            jax.ShapeDtypeStruct((NB * 128,), jnp.int32),
            jax.ShapeDtypeStruct((NB * 512,), jnp.float32),
        ),
        mesh=mesh,
        compiler_params=pltpu.CompilerParams(needs_layout_passes=False),
        scratch_types=[
            pltpu.VMEM((N_PAD,), jnp.int32),
            pltpu.VMEM((CAP_A,), jnp.int32),
            pltpu.VMEM((CAP_A,), jnp.int32),
            pltpu.VMEM((CAP_B,), jnp.int32),
            pltpu.VMEM((CAP_B,), jnp.int32),
            pltpu.VMEM((4352,), jnp.int32),
            pltpu.VMEM((256,), jnp.int32),
            pltpu.VMEM((160,), jnp.int32),
            pltpu.VMEM((160,), jnp.int32),
            pltpu.VMEM((128,), jnp.int32),
            pltpu.VMEM((128,), jnp.int32),
            pltpu.VMEM((3616,), jnp.float32),
            pltpu.VMEM((32,), jnp.float32),
            pltpu.VMEM((128,), jnp.float32),
            pltpu.VMEM((128,), jnp.int32),
            pltpu.VMEM((512,), jnp.float32),
        ],
    )
    return f(keys_flat, boxes_flat, scale_flat)


def kernel(pred_logits, pred_boxes, target_sizes):
    B, Q, C = pred_logits.shape
    flat = pred_logits.reshape(B, Q * C)
    bits = lax.bitcast_convert_type(flat, jnp.int32)
    keys = jnp.where(bits >= 0, bits, bits ^ 0x7FFFFFFF)
    keys = jnp.pad(keys, ((0, 0), (0, N_PAD - N_REAL)),
                   constant_values=INT_MIN)
    ts = target_sizes.astype(jnp.float32)
    scale = jnp.concatenate(
        [jnp.tile(ts[:, 1:2], (1, 16)), jnp.tile(ts[:, 0:1], (1, 16))], axis=1)
    sc_f, lb_f, bx_f = _post_process_sc(
        keys.reshape(-1), pred_boxes.reshape(-1), scale.reshape(-1))
    scores = sc_f.reshape(B, 128)[:, :K_SEL]
    labels = lb_f.reshape(B, 128)[:, :K_SEL]
    boxes = bx_f.reshape(B, 4, 128)[:, :, :K_SEL].transpose(0, 2, 1)
    return scores, labels, boxes


# speedup vs baseline: 6.0506x; 1.9461x over previous
import functools

import jax
import jax.numpy as jnp
from jax import lax
from jax.experimental import pallas as pl
from jax.experimental.pallas import tpu as pltpu
from jax.experimental.pallas import tpu_sc as plsc

K_SEL = 120
N_REAL = 81900
N_PAD = 81920
NB = 128
QN = 900
CN = 91
NW = 32
BPW = NB // NW
CAP_A = 8208
CAP_B = 4112
INT_MIN = -(2 ** 31)
UH = 8
NV = N_PAD // 16


def _sc_body(keys_hbm, boxes_hbm, scale_hbm, sc_out, lb_out, bx_out,
             data, ca_k, ca_i, cb_k, cb_i, histL, hist1,
             win_k, win_i, srt_k, srt_i, boxbuf, scalebuf, cnts,
             sc_buf, lb_buf, bx_buf):
    wid = lax.axis_index("s") * 2 + lax.axis_index("c")
    iota = lax.broadcasted_iota(jnp.int32, (16,), 0)
    iota17 = iota * 17
    zeros16 = jnp.zeros((16,), jnp.int32)
    ones16 = jnp.ones((16,), jnp.int32)

    def zero_hist():
        def zh(m, c):
            for u in range(UH):
                histL[pl.ds((m * UH + u) * 16, 16)] = zeros16
            return c
        lax.fori_loop(0, 4352 // (16 * UH), zh, 0)

    def select_digit(k_need):
        def rb(g, c):
            acc = zeros16
            gbase = g * 272 + iota17
            for j in range(16):
                acc = acc + plsc.load_gather(histL, [gbase + j])
            hist1[pl.ds(g * 16, 16)] = acc
            return c
        lax.fori_loop(0, 16, rb, 0)

        def sb(t, carry):
            bsel, acc = carry
            g = 15 - t
            v = hist1[pl.ds(g * 16, 16)]
            suff = lax.rev(plsc.cumsum(lax.rev(v, (0,))), (0,))
            cnt_ge = acc + suff
            dv = jnp.where(cnt_ge >= k_need, g * 16 + iota, -1)
            bsel = jnp.maximum(bsel, jnp.max(dv))
            return bsel, acc + jnp.sum(v)
        bsel, _ = lax.fori_loop(0, 16, sb, (jnp.int32(-1), jnp.int32(0)))

        def cgb(g, acc):
            v = hist1[pl.ds(g * 16, 16)]
            return acc + jnp.sum(jnp.where((g * 16 + iota) > bsel, v, 0))
        cntgt = lax.fori_loop(0, 16, cgb, jnp.int32(0))
        return bsel, cntgt

    def mid_round(src_k, n_src, shift, base, k_r):
        zero_hist()
        himax = base + ((1 << (shift + 8)) - 1)

        def hb(i, c):
            kv = src_k[pl.ds(i * 16, 16)]
            m = (kv <= himax) & ((i * 16 + iota) < n_src)
            d = (kv >> shift) & 0xFF
            plsc.addupdate_scatter(histL, [d * 17 + iota], ones16, mask=m)
            return c
        lax.fori_loop(0, (n_src + 15) // 16, hb, 0)
        bsel, cg = select_digit(k_r)
        return base + (bsel << shift), cg

    def mid_collect(src_k, src_i, dst_k, dst_i, n_src, base, cap):
        def cb(i, coff):
            kv = src_k[pl.ds(i * 16, 16)]
            iv = src_i[pl.ds(i * 16, 16)]
            em = ((kv >= base) & ((i * 16 + iota) < n_src)
                  & (coff < cap - 16))
            plsc.store_compressed(dst_k.at[pl.ds(coff, 16)], kv, mask=em)
            plsc.store_compressed(dst_i.at[pl.ds(coff, 16)], iv, mask=em)
            return coff + jnp.sum(em)
        return lax.fori_loop(0, (n_src + 15) // 16, cb, jnp.int32(0))

    def per_batch(t, carry):
        b = wid * BPW + t
        pltpu.sync_copy(keys_hbm.at[pl.ds(b * N_PAD, N_PAD)], data)
        pltpu.sync_copy(boxes_hbm.at[pl.ds(b * (QN * 4), QN * 4)],
                        boxbuf.at[pl.ds(0, QN * 4)])
        pltpu.sync_copy(scale_hbm.at[pl.ds(b * 32, 32)], scalebuf)

        win_k[pl.ds(112, 16)] = jnp.full((16,), INT_MIN, jnp.int32)
        win_i[pl.ds(112, 16)] = N_PAD + iota

        zero_hist()

        @plsc.parallel_loop(0, NV, 1, unroll=UH)
        def _h1(i):
            kv = data[pl.ds(i * 16, 16)]
            d = (kv >> 24) + 128
            plsc.addupdate_scatter(histL, [d * 17 + iota], ones16)

        b1, cg1 = select_digit(jnp.int32(K_SEL))
        base = (b1 - 128) << 24

        @plsc.parallel_loop(0, NV, 1, unroll=UH)
        def _p1(i):
            em = data[pl.ds(i * 16, 16)] >= base
            pc = plsc.all_reduce_population_count(em)
            plsc.store_scatter(cnts, [iota * 0 + i], pc, mask=(iota == 0))

        def pb(g, acc):
            v = cnts[pl.ds(g * 16, 16)]
            incl = plsc.cumsum(v)
            cnts[pl.ds(g * 16, 16)] = incl - v + acc
            return acc + incl[15]
        c1n = lax.fori_loop(0, NV // 16, pb, jnp.int32(0))
        c1n = jnp.minimum(c1n, CAP_A - 16)

        @plsc.parallel_loop(0, NV, 1, unroll=UH)
        def _p2(i):
            kv = data[pl.ds(i * 16, 16)]
            coff = cnts[pl.ds(i, 16)][0]
            em = (kv >= base) & (coff < CAP_A - 16)
            plsc.store_compressed(ca_k.at[pl.ds(coff, 16)], kv, mask=em)
            plsc.store_compressed(ca_i.at[pl.ds(coff, 16)], i * 16 + iota,
                                  mask=em)

        k2 = K_SEL - cg1
        base, cg2 = mid_round(ca_k, c1n, 16, base, k2)
        c2n = mid_collect(ca_k, ca_i, cb_k, cb_i, c1n, base, CAP_B)
        k3 = k2 - cg2
        base, cg3 = mid_round(cb_k, c2n, 8, base, k3)
        c3n = mid_collect(cb_k, cb_i, ca_k, ca_i, c2n, base, CAP_A)
        k4 = k3 - cg3
        tkey, cg4 = mid_round(ca_k, c3n, 0, base, k4)
        k5 = k4 - cg4

        def fb(i, carry):
            woff, cseen = carry
            kv = ca_k[pl.ds(i * 16, 16)]
            iv = ca_i[pl.ds(i * 16, 16)]
            valid = (i * 16 + iota) < c3n
            wm = (kv > tkey) & valid
            plsc.store_compressed(win_k.at[pl.ds(woff, 16)], kv, mask=wm)
            plsc.store_compressed(win_i.at[pl.ds(woff, 16)], iv, mask=wm)
            woff = woff + jnp.sum(wm)
            em = (kv == tkey) & valid
            r = cseen + plsc.cumsum(em.astype(jnp.int32)) - 1
            tm = em & (r < k5)
            plsc.store_compressed(win_k.at[pl.ds(woff, 16)], kv, mask=tm)
            plsc.store_compressed(win_i.at[pl.ds(woff, 16)], iv, mask=tm)
            return woff + jnp.sum(tm), cseen + jnp.sum(em)
        lax.fori_loop(0, (c3n + 15) // 16, fb,
                      (jnp.int32(0), jnp.int32(0)))

        def rank_i(i, c):
            ki = win_k[pl.ds(i * 16, 16)]
            ii = win_i[pl.ds(i * 16, 16)]

            def rank_m(m, acc):
                for u in range(4):
                    mm = m * 4 + u
                    idxv = ((mm >> 4) << 4) + ((iota + mm) & 15)
                    kr = plsc.load_gather(win_k, [idxv])
                    ir = plsc.load_gather(win_i, [idxv])
                    beat = (kr > ki) | ((kr == ki) & (ir < ii))
                    acc = acc + beat.astype(jnp.int32)
                return acc
            pos = lax.fori_loop(0, 32, rank_m, zeros16)
            plsc.store_scatter(srt_k, [pos], ki)
            plsc.store_scatter(srt_i, [pos], ii)
            return c
        lax.fori_loop(0, 8, rank_i, 0)

        W = scalebuf[pl.ds(0, 16)]
        H = scalebuf[pl.ds(16, 16)]

        def out_v(v, c):
            kk = srt_k[pl.ds(v * 16, 16)]
            id2 = srt_i[pl.ds(v * 16, 16)]
            bits = jnp.where(kk >= 0, kk, kk ^ 0x7FFFFFFF)
            val = plsc.bitcast(bits, jnp.float32)
            sc_buf[pl.ds(v * 16, 16)] = 1.0 / (1.0 + jnp.exp(-val))
            lb_buf[pl.ds(v * 16, 16)] = id2 % CN
            q4 = jnp.minimum(id2 // CN, QN - 1) * 4
            cx = plsc.load_gather(boxbuf, [q4])
            cy = plsc.load_gather(boxbuf, [q4 + 1])
            w = plsc.load_gather(boxbuf, [q4 + 2])
            h = plsc.load_gather(boxbuf, [q4 + 3])
            bx_buf[pl.ds(v * 16, 16)] = (cx - 0.5 * w) * W
            bx_buf[pl.ds(128 + v * 16, 16)] = (cy - 0.5 * h) * H
            bx_buf[pl.ds(256 + v * 16, 16)] = (cx + 0.5 * w) * W
            bx_buf[pl.ds(384 + v * 16, 16)] = (cy + 0.5 * h) * H
            return c
        lax.fori_loop(0, 8, out_v, 0)

        pltpu.sync_copy(sc_buf, sc_out.at[pl.ds(b * 128, 128)])
        pltpu.sync_copy(lb_buf, lb_out.at[pl.ds(b * 128, 128)])
        pltpu.sync_copy(bx_buf, bx_out.at[pl.ds(b * 512, 512)])
        return carry

    lax.fori_loop(0, BPW, per_batch, 0)


@jax.jit
def _post_process_sc(keys_flat, boxes_flat, scale_flat):
    mesh = plsc.VectorSubcoreMesh(core_axis_name="c", subcore_axis_name="s")
    f = pl.kernel(
        _sc_body,
        out_type=(
            jax.ShapeDtypeStruct((NB * 128,), jnp.float32),
            jax.ShapeDtypeStruct((NB * 128,), jnp.int32),
            jax.ShapeDtypeStruct((NB * 512,), jnp.float32),
        ),
        mesh=mesh,
        compiler_params=pltpu.CompilerParams(needs_layout_passes=False),
        scratch_types=[
            pltpu.VMEM((N_PAD,), jnp.int32),
            pltpu.VMEM((CAP_A,), jnp.int32),
            pltpu.VMEM((CAP_A,), jnp.int32),
            pltpu.VMEM((CAP_B,), jnp.int32),
            pltpu.VMEM((CAP_B,), jnp.int32),
            pltpu.VMEM((4352,), jnp.int32),
            pltpu.VMEM((256,), jnp.int32),
            pltpu.VMEM((160,), jnp.int32),
            pltpu.VMEM((160,), jnp.int32),
            pltpu.VMEM((128,), jnp.int32),
            pltpu.VMEM((128,), jnp.int32),
            pltpu.VMEM((3616,), jnp.float32),
            pltpu.VMEM((32,), jnp.float32),
            pltpu.VMEM((NV + 16,), jnp.int32),
            pltpu.VMEM((128,), jnp.float32),
            pltpu.VMEM((128,), jnp.int32),
            pltpu.VMEM((512,), jnp.float32),
        ],
    )
    return f(keys_flat, boxes_flat, scale_flat)


def kernel(pred_logits, pred_boxes, target_sizes):
    B, Q, C = pred_logits.shape
    flat = pred_logits.reshape(B, Q * C)
    bits = lax.bitcast_convert_type(flat, jnp.int32)
    keys = jnp.where(bits >= 0, bits, bits ^ 0x7FFFFFFF)
    keys = jnp.pad(keys, ((0, 0), (0, N_PAD - N_REAL)),
                   constant_values=INT_MIN)
    ts = target_sizes.astype(jnp.float32)
    scale = jnp.concatenate(
        [jnp.tile(ts[:, 1:2], (1, 16)), jnp.tile(ts[:, 0:1], (1, 16))], axis=1)
    sc_f, lb_f, bx_f = _post_process_sc(
        keys.reshape(-1), pred_boxes.reshape(-1), scale.reshape(-1))
    scores = sc_f.reshape(B, 128)[:, :K_SEL]
    labels = lb_f.reshape(B, 128)[:, :K_SEL]
    boxes = bx_f.reshape(B, 4, 128)[:, :, :K_SEL].transpose(0, 2, 1)
    return scores, labels, boxes
